# baseline (speedup 1.0000x reference)
import functools

import jax
import jax.numpy as jnp
from jax import lax
from jax.experimental import pallas as pl
from jax.experimental.pallas import tpu as pltpu
from jax.experimental.pallas import tpu_sc as plsc

N_FIELDS = 26
VOCAB = 100000
EMBED_DIM = 32
BATCH = 16384

NC = 2
NS = 16
L = 16
NW = NC * NS

ROWS = BATCH * N_FIELDS
RPW = ROWS // NW
GCHUNK = 128
OCHUNK = 1024
NGO = OCHUNK // GCHUNK
NOUTER = RPW // OCHUNK

VB = 1024
NVB = (VOCAB + VB - 1) // VB


def _tc_detile(tab_t2):
    def body(i_ref, o_ref):
        eye = (
            lax.broadcasted_iota(jnp.int32, (EMBED_DIM, EMBED_DIM), 0)
            == lax.broadcasted_iota(jnp.int32, (EMBED_DIM, EMBED_DIM), 1)
        ).astype(jnp.float32)
        o_ref[0] = lax.dot_general(
            i_ref[...],
            eye,
            (((0,), (0,)), ((), ())),
            preferred_element_type=jnp.float32,
        )

    return pl.pallas_call(
        body,
        grid=(N_FIELDS, NVB),
        in_specs=[
            pl.BlockSpec((EMBED_DIM, VB), lambda f, k: (f, k)),
        ],
        out_specs=pl.BlockSpec((1, VB, EMBED_DIM), lambda f, k: (f, k, 0)),
        out_shape=jax.ShapeDtypeStruct(
            (N_FIELDS, VOCAB, EMBED_DIM), jnp.float32
        ),
    )(tab_t2)


def _sc_gather(idx_flat, table2d):
    mesh = plsc.VectorSubcoreMesh(core_axis_name="c", subcore_axis_name="s")

    @functools.partial(
        pl.kernel,
        out_type=jax.ShapeDtypeStruct((ROWS, EMBED_DIM), jnp.float32),
        mesh=mesh,
        scratch_types=[
            pltpu.VMEM((RPW,), jnp.int32),
            pltpu.VMEM((OCHUNK, EMBED_DIM), jnp.float32),
            pltpu.SemaphoreType.DMA,
        ],
        compiler_params=pltpu.CompilerParams(use_tc_tiling_on_sc=False),
    )
    def k(idx_hbm, table_hbm, out_hbm, idx_v, rows_v, sem):
        wid = lax.axis_index("s") * NC + lax.axis_index("c")
        base = wid * RPW

        pltpu.sync_copy(idx_hbm.at[pl.ds(base, RPW)], idx_v)

        lane = lax.iota(jnp.int32, L)

        def fix(i, carry):
            p = i * L + lane
            f = lax.rem(p, N_FIELDS)
            sl = pl.ds(i * L, L)
            idx_v[sl] = idx_v[sl] + f * VOCAB
            return carry

        lax.fori_loop(0, RPW // L, fix, 0)

        def outer(c, carry):
            row0 = c * OCHUNK
            copies = []
            for g in range(NGO):
                src = table_hbm.at[idx_v.at[pl.ds(row0 + g * GCHUNK, GCHUNK)]]
                dst = rows_v.at[pl.ds(g * GCHUNK, GCHUNK), :]
                copies.append(pltpu.async_copy(src, dst, sem))
            for cp in copies:
                cp.wait()
            pltpu.sync_copy(rows_v, out_hbm.at[pl.ds(base + row0, OCHUNK), :])
            return carry

        lax.fori_loop(0, NOUTER, outer, 0)

    return k(idx_flat, table2d)


def kernel(sparse_inputs, tables):
    idx = sparse_inputs.astype(jnp.int32).reshape(ROWS)
    tab_t2 = jnp.transpose(tables, (0, 2, 1)).reshape(
        N_FIELDS * EMBED_DIM, VOCAB
    )
    tab_lin = _tc_detile(tab_t2).reshape(N_FIELDS * VOCAB, EMBED_DIM)
    out = _sc_gather(idx, tab_lin)
    return out.reshape(BATCH, N_FIELDS, EMBED_DIM)


# speedup vs baseline: 1.6779x; 1.6779x over previous
import functools

import jax
import jax.numpy as jnp
from jax import lax
from jax.experimental import pallas as pl
from jax.experimental.pallas import tpu as pltpu
from jax.experimental.pallas import tpu_sc as plsc

N_FIELDS = 26
VOCAB = 100000
EMBED_DIM = 32
BATCH = 16384

NC = 2
NS = 16
L = 16
NW = NC * NS

ROWS = BATCH * N_FIELDS
RPW = ROWS // NW
GCHUNK = 128
OCHUNK = 1024
NGO = OCHUNK // GCHUNK
NOUTER = RPW // OCHUNK

VB = 8192
NVB = (VOCAB + VB - 1) // VB


def _tc_detile(tab_t2):
    def body(i_ref, o_ref):
        o_ref[0] = i_ref[...].T

    return pl.pallas_call(
        body,
        grid=(N_FIELDS, NVB),
        in_specs=[
            pl.BlockSpec((EMBED_DIM, VB), lambda f, k: (f, k)),
        ],
        out_specs=pl.BlockSpec((1, VB, EMBED_DIM), lambda f, k: (f, k, 0)),
        out_shape=jax.ShapeDtypeStruct(
            (N_FIELDS, VOCAB, EMBED_DIM), jnp.float32
        ),
    )(tab_t2)


def _sc_gather(idx_flat, table2d):
    mesh = plsc.VectorSubcoreMesh(core_axis_name="c", subcore_axis_name="s")

    @functools.partial(
        pl.kernel,
        out_type=jax.ShapeDtypeStruct((ROWS, EMBED_DIM), jnp.float32),
        mesh=mesh,
        scratch_types=[
            pltpu.VMEM((RPW,), jnp.int32),
            pltpu.VMEM((OCHUNK, EMBED_DIM), jnp.float32),
            pltpu.SemaphoreType.DMA,
        ],
        compiler_params=pltpu.CompilerParams(use_tc_tiling_on_sc=False),
    )
    def k(idx_hbm, table_hbm, out_hbm, idx_v, rows_v, sem):
        wid = lax.axis_index("s") * NC + lax.axis_index("c")
        base = wid * RPW

        pltpu.sync_copy(idx_hbm.at[pl.ds(base, RPW)], idx_v)

        lane = lax.iota(jnp.int32, L)

        def fix(i, carry):
            p = i * L + lane
            f = lax.rem(p, N_FIELDS)
            sl = pl.ds(i * L, L)
            idx_v[sl] = idx_v[sl] + f * VOCAB
            return carry

        lax.fori_loop(0, RPW // L, fix, 0)

        def outer(c, carry):
            row0 = c * OCHUNK
            copies = []
            for g in range(NGO):
                src = table_hbm.at[idx_v.at[pl.ds(row0 + g * GCHUNK, GCHUNK)]]
                dst = rows_v.at[pl.ds(g * GCHUNK, GCHUNK), :]
                copies.append(pltpu.async_copy(src, dst, sem))
            for cp in copies:
                cp.wait()
            pltpu.sync_copy(rows_v, out_hbm.at[pl.ds(base + row0, OCHUNK), :])
            return carry

        lax.fori_loop(0, NOUTER, outer, 0)

    return k(idx_flat, table2d)


def kernel(sparse_inputs, tables):
    idx = sparse_inputs.astype(jnp.int32).reshape(ROWS)
    tab_t2 = jnp.transpose(tables, (0, 2, 1)).reshape(
        N_FIELDS * EMBED_DIM, VOCAB
    )
    tab_lin = _tc_detile(tab_t2).reshape(N_FIELDS * VOCAB, EMBED_DIM)
    out = _sc_gather(idx, tab_lin)
    return out.reshape(BATCH, N_FIELDS, EMBED_DIM)


# speedup vs baseline: 4.4188x; 2.6335x over previous
import functools

import jax
import jax.numpy as jnp
from jax import lax
from jax.experimental import pallas as pl
from jax.experimental.pallas import tpu as pltpu
from jax.experimental.pallas import tpu_sc as plsc

N_FIELDS = 26
VOCAB = 100000
EMBED_DIM = 32
BATCH = 16384

NC = 2
NS = 16
L = 16
NW = NC * NS

ROWS = BATCH * N_FIELDS
RPW = ROWS // NW
GCHUNK = 128
OCHUNK = 1024
NGO = OCHUNK // GCHUNK
NOUTER = RPW // OCHUNK

VB = 2048
NVB = (VOCAB + VB - 1) // VB
FQ = 4
NQ = (N_FIELDS + FQ - 1) // FQ
QW = FQ * EMBED_DIM


def _tc_detile(tab_t2):
    def body(i_ref, o_ref):
        o_ref[0] = i_ref[...].T

    return pl.pallas_call(
        body,
        grid=(NQ, NVB),
        in_specs=[
            pl.BlockSpec((QW, VB), lambda q, k: (q, k)),
        ],
        out_specs=pl.BlockSpec((1, VB, QW), lambda q, k: (q, k, 0)),
        out_shape=jax.ShapeDtypeStruct((NQ, VOCAB, QW), jnp.float32),
    )(tab_t2)


def _sc_gather(idx_flat, table2d):
    mesh = plsc.VectorSubcoreMesh(core_axis_name="c", subcore_axis_name="s")

    @functools.partial(
        pl.kernel,
        out_type=jax.ShapeDtypeStruct((ROWS, EMBED_DIM), jnp.float32),
        mesh=mesh,
        scratch_types=[
            pltpu.VMEM((RPW,), jnp.int32),
            pltpu.VMEM((OCHUNK, EMBED_DIM), jnp.float32),
            pltpu.SemaphoreType.DMA,
        ],
        compiler_params=pltpu.CompilerParams(use_tc_tiling_on_sc=False),
    )
    def k(idx_hbm, table_hbm, out_hbm, idx_v, rows_v, sem):
        wid = lax.axis_index("s") * NC + lax.axis_index("c")
        base = wid * RPW

        pltpu.sync_copy(idx_hbm.at[pl.ds(base, RPW)], idx_v)

        lane = lax.iota(jnp.int32, L)

        def fix(i, carry):
            p = i * L + lane
            f = lax.rem(p, N_FIELDS)
            sl = pl.ds(i * L, L)
            idx_v[sl] = (
                idx_v[sl] * FQ
                + lax.div(f, FQ) * (FQ * VOCAB)
                + lax.rem(f, FQ)
            )
            return carry

        lax.fori_loop(0, RPW // L, fix, 0)

        def outer(c, carry):
            row0 = c * OCHUNK
            copies = []
            for g in range(NGO):
                src = table_hbm.at[idx_v.at[pl.ds(row0 + g * GCHUNK, GCHUNK)]]
                dst = rows_v.at[pl.ds(g * GCHUNK, GCHUNK), :]
                copies.append(pltpu.async_copy(src, dst, sem))
            for cp in copies:
                cp.wait()
            pltpu.sync_copy(rows_v, out_hbm.at[pl.ds(base + row0, OCHUNK), :])
            return carry

        lax.fori_loop(0, NOUTER, outer, 0)

    return k(idx_flat, table2d)


def kernel(sparse_inputs, tables):
    idx = sparse_inputs.astype(jnp.int32).reshape(ROWS)
    tab_t2 = jnp.transpose(tables, (0, 2, 1)).reshape(
        N_FIELDS * EMBED_DIM, VOCAB
    )
    tab_lin = _tc_detile(tab_t2).reshape(NQ * VOCAB * FQ, EMBED_DIM)
    out = _sc_gather(idx, tab_lin)
    return out.reshape(BATCH, N_FIELDS, EMBED_DIM)


# speedup vs baseline: 5.1472x; 1.1649x over previous
import functools

import jax
import jax.numpy as jnp
from jax import lax
from jax.experimental import pallas as pl
from jax.experimental.pallas import tpu as pltpu
from jax.experimental.pallas import tpu_sc as plsc

N_FIELDS = 26
VOCAB = 100000
EMBED_DIM = 32
BATCH = 16384

NC = 2
NS = 16
L = 16
NW = NC * NS

ROWS = BATCH * N_FIELDS
RPW = ROWS // NW
GCHUNK = 128
OCHUNK = 1024
NGO = OCHUNK // GCHUNK
NOUTER = RPW // OCHUNK

VB = 4096
NVB = (VOCAB + VB - 1) // VB
FQ = 4
NQ = (N_FIELDS + FQ - 1) // FQ
QW = FQ * EMBED_DIM


def _tc_detile(tab_t2):
    def body(i_ref, o_ref):
        o_ref[0] = i_ref[...].T

    return pl.pallas_call(
        body,
        grid=(NQ, NVB),
        in_specs=[
            pl.BlockSpec((QW, VB), lambda q, k: (q, k)),
        ],
        out_specs=pl.BlockSpec((1, VB, QW), lambda q, k: (q, k, 0)),
        out_shape=jax.ShapeDtypeStruct((NQ, VOCAB, QW), jnp.float32),
    )(tab_t2)


def _sc_gather(idx_flat, table2d):
    mesh = plsc.VectorSubcoreMesh(core_axis_name="c", subcore_axis_name="s")

    @functools.partial(
        pl.kernel,
        out_type=jax.ShapeDtypeStruct((ROWS, EMBED_DIM), jnp.float32),
        mesh=mesh,
        scratch_types=[
            pltpu.VMEM((RPW,), jnp.int32),
            pltpu.VMEM((OCHUNK, EMBED_DIM), jnp.float32),
            pltpu.SemaphoreType.DMA,
        ],
        compiler_params=pltpu.CompilerParams(use_tc_tiling_on_sc=False),
    )
    def k(idx_hbm, table_hbm, out_hbm, idx_v, rows_v, sem):
        wid = lax.axis_index("s") * NC + lax.axis_index("c")
        base = wid * RPW

        pltpu.sync_copy(idx_hbm.at[pl.ds(base, RPW)], idx_v)

        lane = lax.iota(jnp.int32, L)

        def fix(i, carry):
            p = i * L + lane
            f = lax.rem(p, N_FIELDS)
            sl = pl.ds(i * L, L)
            idx_v[sl] = (
                idx_v[sl] * FQ
                + lax.div(f, FQ) * (FQ * VOCAB)
                + lax.rem(f, FQ)
            )
            return carry

        lax.fori_loop(0, RPW // L, fix, 0)

        def outer(c, carry):
            row0 = c * OCHUNK
            copies = []
            for g in range(NGO):
                src = table_hbm.at[idx_v.at[pl.ds(row0 + g * GCHUNK, GCHUNK)]]
                dst = rows_v.at[pl.ds(g * GCHUNK, GCHUNK), :]
                copies.append(pltpu.async_copy(src, dst, sem))
            for cp in copies:
                cp.wait()
            pltpu.sync_copy(rows_v, out_hbm.at[pl.ds(base + row0, OCHUNK), :])
            return carry

        lax.fori_loop(0, NOUTER, outer, 0)

    return k(idx_flat, table2d)


def kernel(sparse_inputs, tables):
    idx = sparse_inputs.astype(jnp.int32).reshape(ROWS)
    tab_t2 = jnp.transpose(tables, (0, 2, 1)).reshape(
        N_FIELDS * EMBED_DIM, VOCAB
    )
    tab_lin = _tc_detile(tab_t2).reshape(NQ * VOCAB * FQ, EMBED_DIM)
    out = _sc_gather(idx, tab_lin)
    return out.reshape(BATCH, N_FIELDS, EMBED_DIM)


# speedup vs baseline: 5.4929x; 1.0671x over previous
import functools

import jax
import jax.numpy as jnp
from jax import lax
from jax.experimental import pallas as pl
from jax.experimental.pallas import tpu as pltpu
from jax.experimental.pallas import tpu_sc as plsc

N_FIELDS = 26
VOCAB = 100000
EMBED_DIM = 32
BATCH = 16384

NC = 2
NS = 16
L = 16
NW = NC * NS

ROWS = BATCH * N_FIELDS
RPW = ROWS // NW
GCHUNK = 128
OCHUNK = 1024
NGO = OCHUNK // GCHUNK
NOUTER = RPW // OCHUNK

VB = 8192
NVB = (VOCAB + VB - 1) // VB
FQ = 4
NQ = (N_FIELDS + FQ - 1) // FQ
QW = FQ * EMBED_DIM


def _tc_detile(tab_t2):
    def body(i_ref, o_ref):
        o_ref[0] = i_ref[...].T

    return pl.pallas_call(
        body,
        grid=(NQ, NVB),
        in_specs=[
            pl.BlockSpec((QW, VB), lambda q, k: (q, k)),
        ],
        out_specs=pl.BlockSpec((1, VB, QW), lambda q, k: (q, k, 0)),
        out_shape=jax.ShapeDtypeStruct((NQ, VOCAB, QW), jnp.float32),
    )(tab_t2)


def _sc_gather(idx_flat, table2d):
    mesh = plsc.VectorSubcoreMesh(core_axis_name="c", subcore_axis_name="s")

    @functools.partial(
        pl.kernel,
        out_type=jax.ShapeDtypeStruct((ROWS, EMBED_DIM), jnp.float32),
        mesh=mesh,
        scratch_types=[
            pltpu.VMEM((RPW,), jnp.int32),
            pltpu.VMEM((OCHUNK, EMBED_DIM), jnp.float32),
            pltpu.SemaphoreType.DMA,
        ],
        compiler_params=pltpu.CompilerParams(use_tc_tiling_on_sc=False),
    )
    def k(idx_hbm, table_hbm, out_hbm, idx_v, rows_v, sem):
        wid = lax.axis_index("s") * NC + lax.axis_index("c")
        base = wid * RPW

        pltpu.sync_copy(idx_hbm.at[pl.ds(base, RPW)], idx_v)

        lane = lax.iota(jnp.int32, L)

        def fix(i, carry):
            p = i * L + lane
            f = lax.rem(p, N_FIELDS)
            sl = pl.ds(i * L, L)
            idx_v[sl] = (
                idx_v[sl] * FQ
                + lax.div(f, FQ) * (FQ * VOCAB)
                + lax.rem(f, FQ)
            )
            return carry

        lax.fori_loop(0, RPW // L, fix, 0)

        def outer(c, carry):
            row0 = c * OCHUNK
            copies = []
            for g in range(NGO):
                src = table_hbm.at[idx_v.at[pl.ds(row0 + g * GCHUNK, GCHUNK)]]
                dst = rows_v.at[pl.ds(g * GCHUNK, GCHUNK), :]
                copies.append(pltpu.async_copy(src, dst, sem))
            for cp in copies:
                cp.wait()
            pltpu.sync_copy(rows_v, out_hbm.at[pl.ds(base + row0, OCHUNK), :])
            return carry

        lax.fori_loop(0, NOUTER, outer, 0)

    return k(idx_flat, table2d)


def kernel(sparse_inputs, tables):
    idx = sparse_inputs.astype(jnp.int32).reshape(ROWS)
    tab_t2 = jnp.transpose(tables, (0, 2, 1)).reshape(
        N_FIELDS * EMBED_DIM, VOCAB
    )
    tab_lin = _tc_detile(tab_t2).reshape(NQ * VOCAB * FQ, EMBED_DIM)
    out = _sc_gather(idx, tab_lin)
    return out.reshape(BATCH, N_FIELDS, EMBED_DIM)


# speedup vs baseline: 5.5535x; 1.0110x over previous
import functools

import jax
import jax.numpy as jnp
from jax import lax
from jax.experimental import pallas as pl
from jax.experimental.pallas import tpu as pltpu
from jax.experimental.pallas import tpu_sc as plsc

N_FIELDS = 26
VOCAB = 100000
EMBED_DIM = 32
BATCH = 16384

NC = 2
NS = 16
L = 16
NW = NC * NS

ROWS = BATCH * N_FIELDS
RPW = ROWS // NW
GCHUNK = 128
OCHUNK = 1024
NGO = OCHUNK // GCHUNK
NOUTER = RPW // OCHUNK

VB = 16384
NVB = (VOCAB + VB - 1) // VB
FQ = 4
NQ = (N_FIELDS + FQ - 1) // FQ
QW = FQ * EMBED_DIM


def _tc_detile(tab_t2):
    def body(i_ref, o_ref):
        o_ref[0] = i_ref[...].T

    return pl.pallas_call(
        body,
        grid=(NQ, NVB),
        in_specs=[
            pl.BlockSpec((QW, VB), lambda q, k: (q, k)),
        ],
        out_specs=pl.BlockSpec((1, VB, QW), lambda q, k: (q, k, 0)),
        out_shape=jax.ShapeDtypeStruct((NQ, VOCAB, QW), jnp.float32),
    )(tab_t2)


def _sc_gather(idx_flat, table2d):
    mesh = plsc.VectorSubcoreMesh(core_axis_name="c", subcore_axis_name="s")

    @functools.partial(
        pl.kernel,
        out_type=jax.ShapeDtypeStruct((ROWS, EMBED_DIM), jnp.float32),
        mesh=mesh,
        scratch_types=[
            pltpu.VMEM((RPW,), jnp.int32),
            pltpu.VMEM((OCHUNK, EMBED_DIM), jnp.float32),
            pltpu.SemaphoreType.DMA,
        ],
        compiler_params=pltpu.CompilerParams(use_tc_tiling_on_sc=False),
    )
    def k(idx_hbm, table_hbm, out_hbm, idx_v, rows_v, sem):
        wid = lax.axis_index("s") * NC + lax.axis_index("c")
        base = wid * RPW

        pltpu.sync_copy(idx_hbm.at[pl.ds(base, RPW)], idx_v)

        lane = lax.iota(jnp.int32, L)

        def fix(i, carry):
            p = i * L + lane
            f = lax.rem(p, N_FIELDS)
            sl = pl.ds(i * L, L)
            idx_v[sl] = (
                idx_v[sl] * FQ
                + lax.div(f, FQ) * (FQ * VOCAB)
                + lax.rem(f, FQ)
            )
            return carry

        lax.fori_loop(0, RPW // L, fix, 0)

        def outer(c, carry):
            row0 = c * OCHUNK
            copies = []
            for g in range(NGO):
                src = table_hbm.at[idx_v.at[pl.ds(row0 + g * GCHUNK, GCHUNK)]]
                dst = rows_v.at[pl.ds(g * GCHUNK, GCHUNK), :]
                copies.append(pltpu.async_copy(src, dst, sem))
            for cp in copies:
                cp.wait()
            pltpu.sync_copy(rows_v, out_hbm.at[pl.ds(base + row0, OCHUNK), :])
            return carry

        lax.fori_loop(0, NOUTER, outer, 0)

    return k(idx_flat, table2d)


def kernel(sparse_inputs, tables):
    idx = sparse_inputs.astype(jnp.int32).reshape(ROWS)
    tab_t2 = jnp.transpose(tables, (0, 2, 1)).reshape(
        N_FIELDS * EMBED_DIM, VOCAB
    )
    tab_lin = _tc_detile(tab_t2).reshape(NQ * VOCAB * FQ, EMBED_DIM)
    out = _sc_gather(idx, tab_lin)
    return out.reshape(BATCH, N_FIELDS, EMBED_DIM)
